# baseline (speedup 1.0000x reference)
import functools

import jax
import jax.numpy as jnp
from jax import lax
from jax.experimental import pallas as pl
from jax.experimental.pallas import tpu as pltpu
from jax.experimental.pallas import tpu_sc as plsc

N_VOCAB = 100000
N_CTX = 2048
N_EMBED = 128
BATCH = 4

NC = 2
NS = 16
NW = NC * NS
PPW = N_CTX // NW
LANES = 16


def _sc_embed(x_hbm, wte_hbm, wpe_hbm, out_hbm, idx_v, wv, rows_v, sem_i, sem_p, sem_g, sem_s):
    wid = lax.axis_index("s") * NC + lax.axis_index("c")
    p0 = wid * PPW

    cp_i = [
        pltpu.async_copy(x_hbm.at[b, pl.ds(p0, PPW)], idx_v.at[b], sem_i)
        for b in range(BATCH)
    ]
    cp_w = pltpu.async_copy(wpe_hbm.at[pl.ds(p0, PPW)], wv, sem_p)

    cp_g = []
    for b in range(BATCH):
        cp_i[b].wait()
        cp_g.append(
            pltpu.async_copy(
                wte_hbm.at[idx_v.at[b]],
                rows_v.at[pl.ds(b * PPW, PPW)],
                sem_g,
            )
        )
    cp_w.wait()

    def add_rows(b, r0, nr):
        @pl.loop(r0, r0 + nr, unroll=2)
        def _(r):
            for c in range(N_EMBED // LANES):
                sl = pl.ds(c * LANES, LANES)
                plsc.addupdate(rows_v.at[b * PPW + r, sl], wv[r, sl])

    H = PPW // 2
    cp_s = []
    for b in range(BATCH):
        cp_g[b].wait()
        for h in range(2):
            add_rows(b, h * H, H)
            cp_s.append(
                pltpu.async_copy(
                    rows_v.at[pl.ds(b * PPW + h * H, H)],
                    out_hbm.at[b, pl.ds(p0 + h * H, H)],
                    sem_s,
                )
            )
    for cp in cp_s:
        cp.wait()


@jax.jit
def _embed(x, wte, wpe):
    mesh = plsc.VectorSubcoreMesh(core_axis_name="c", subcore_axis_name="s")
    run = functools.partial(
        pl.kernel,
        out_type=jax.ShapeDtypeStruct((BATCH, N_CTX, N_EMBED), jnp.float32),
        mesh=mesh,
        scratch_types=[
            pltpu.VMEM((BATCH, PPW), jnp.int32),
            pltpu.VMEM((PPW, N_EMBED), jnp.float32),
            pltpu.VMEM((BATCH * PPW, N_EMBED), jnp.float32),
            pltpu.SemaphoreType.DMA,
            pltpu.SemaphoreType.DMA,
            pltpu.SemaphoreType.DMA,
            pltpu.SemaphoreType.DMA,
        ],
    )(_sc_embed)
    return run(x, wte, wpe)


def kernel(x, wte, wpe):
    return _embed(x.astype(jnp.int32), wte, wpe)


# speedup vs baseline: 1.0326x; 1.0326x over previous
import functools

import jax
import jax.numpy as jnp
from jax import lax
from jax.experimental import pallas as pl
from jax.experimental.pallas import tpu as pltpu
from jax.experimental.pallas import tpu_sc as plsc

N_VOCAB = 100000
N_CTX = 2048
N_EMBED = 128
BATCH = 4

NC = 2
NS = 16
NW = NC * NS
PPW = N_CTX // NW
LANES = 16


def _sc_embed(x_hbm, wte_hbm, wpe_hbm, out_hbm, idx_v, wv, rows_v, sem_i, sem_p, sem_g, sem_s):
    wid = lax.axis_index("s") * NC + lax.axis_index("c")
    p0 = wid * PPW

    cp_i = [
        pltpu.async_copy(x_hbm.at[b, pl.ds(p0, PPW)], idx_v.at[b], sem_i)
        for b in range(BATCH)
    ]
    cp_w = pltpu.async_copy(wpe_hbm.at[pl.ds(p0, PPW)], wv, sem_p)

    cp_g = []
    for b in range(BATCH):
        cp_i[b].wait()
        cp_g.append(
            pltpu.async_copy(
                wte_hbm.at[idx_v.at[b]],
                rows_v.at[pl.ds(b * PPW, PPW)],
                sem_g,
            )
        )
    cp_w.wait()

    def add_tile(b):
        @pl.loop(0, PPW)
        def _(r):
            for c in range(N_EMBED // LANES):
                sl = pl.ds(c * LANES, LANES)
                plsc.addupdate(rows_v.at[b * PPW + r, sl], wv[r, sl])

    cp_s = []
    for b in range(BATCH):
        cp_g[b].wait()
        add_tile(b)
        cp_s.append(
            pltpu.async_copy(
                rows_v.at[pl.ds(b * PPW, PPW)],
                out_hbm.at[b, pl.ds(p0, PPW)],
                sem_s,
            )
        )
    for cp in cp_s:
        cp.wait()


@jax.jit
def _embed(x, wte, wpe):
    mesh = plsc.VectorSubcoreMesh(core_axis_name="c", subcore_axis_name="s")
    run = functools.partial(
        pl.kernel,
        out_type=jax.ShapeDtypeStruct((BATCH, N_CTX, N_EMBED), jnp.float32),
        mesh=mesh,
        scratch_types=[
            pltpu.VMEM((BATCH, PPW), jnp.int32),
            pltpu.VMEM((PPW, N_EMBED), jnp.float32),
            pltpu.VMEM((BATCH * PPW, N_EMBED), jnp.float32),
            pltpu.SemaphoreType.DMA,
            pltpu.SemaphoreType.DMA,
            pltpu.SemaphoreType.DMA,
            pltpu.SemaphoreType.DMA,
        ],
    )(_sc_embed)
    return run(x, wte, wpe)


def kernel(x, wte, wpe):
    return _embed(x.astype(jnp.int32), wte, wpe)
